# baseline (device time: 15247 ns/iter reference)
import jax
import jax.numpy as jnp
from jax import lax
from jax.experimental import pallas as pl
from jax.experimental.pallas import tpu as pltpu

B = 8
SKV = 512
H = 8
D = 64
HD = H * D
SCALE = D ** -0.5
NCHUNK = 2
BC = B // NCHUNK


def _hd_mask():
    h2 = lax.broadcasted_iota(jnp.int32, (H, HD), 0)
    j2 = lax.broadcasted_iota(jnp.int32, (H, HD), 1)
    return (j2 // D == h2).astype(jnp.float32)


def kernel(Q, K, V):
    Kt = jnp.transpose(K, (0, 2, 3, 1))
    Vt = jnp.transpose(V, (0, 2, 3, 1))

    def s_body(q_ref, kt_ref, p_ref, stat_ref):
        hd_mask = _hd_mask()
        qb = q_ref[0, 0]
        qrow = jnp.tile(qb, (1, H))
        wt = qrow * hd_mask * SCALE
        ktb = kt_ref[0].reshape(HD, SKV)
        sb = jnp.dot(wt, ktb, preferred_element_type=jnp.float32)
        mb = jnp.max(sb, axis=1, keepdims=True)
        pb = jnp.exp(sb - mb)
        lb = jnp.sum(pb, axis=1, keepdims=True)
        p_ref[0] = pb
        stat_ref[0, :, 0:1] = mb
        stat_ref[0, :, 1:2] = lb

    P, stats = pl.pallas_call(
        s_body,
        grid=(B,),
        out_shape=(
            jax.ShapeDtypeStruct((B, H, SKV), jnp.float32),
            jax.ShapeDtypeStruct((B, H, 2), jnp.float32),
        ),
        in_specs=[
            pl.BlockSpec((1, 1, H, D), lambda b: (b, 0, 0, 0)),
            pl.BlockSpec((1, H, D, SKV), lambda b: (b, 0, 0, 0)),
        ],
        out_specs=(
            pl.BlockSpec((1, H, SKV), lambda b: (b, 0, 0)),
            pl.BlockSpec((1, H, 2), lambda b: (b, 0, 0)),
        ),
    )(Q, Kt)

    def c_body(p_ref, stat_ref, vt_ref, o_ref, comm, send_sems, recv_sems):
        b = pl.program_id(0)
        my_x = lax.axis_index("x")
        my_y = lax.axis_index("y")
        peer = (my_x, 1 - my_y)
        barrier_sem = pltpu.get_barrier_semaphore()

        @pl.when(b == 0)
        def _():
            pl.semaphore_signal(barrier_sem, inc=1, device_id=peer,
                                device_id_type=pl.DeviceIdType.MESH)

        hd_mask = _hd_mask()
        j3 = lax.broadcasted_iota(jnp.int32, (HD, D), 0)
        d3 = lax.broadcasted_iota(jnp.int32, (HD, D), 1)
        e_sum = (j3 % D == d3).astype(jnp.float32)

        pb = p_ref[0]
        vtb = vt_ref[0].reshape(HD, SKV)
        of = lax.dot_general(pb, vtb, (((1,), (1,)), ((), ())),
                             preferred_element_type=jnp.float32)
        ob = jnp.dot(of * hd_mask, e_sum,
                     preferred_element_type=jnp.float32)
        comm[0, b, :, 0:D] = ob
        comm[0, b, :, D:D + 2] = stat_ref[0]

        def rdma_chunk(c):
            lo = c * BC
            return pltpu.make_async_remote_copy(
                src_ref=comm.at[0, lo:lo + BC],
                dst_ref=comm.at[1, lo:lo + BC],
                send_sem=send_sems.at[c], recv_sem=recv_sems.at[c],
                device_id=peer, device_id_type=pl.DeviceIdType.MESH,
            )

        @pl.when(b == BC - 1)
        def _():
            pl.semaphore_wait(barrier_sem, 1)
            rdma_chunk(0).start()

        @pl.when(b == B - 1)
        def _():
            r1 = rdma_chunk(1)
            r1.start()
            rdma_chunk(0).wait()
            r1.wait()

            o_a = comm[0, :, :, 0:D]
            m_a = comm[0, :, :, D:D + 1]
            l_a = comm[0, :, :, D + 1:D + 2]
            o_b = comm[1, :, :, 0:D]
            m_b = comm[1, :, :, D:D + 1]
            l_b = comm[1, :, :, D + 1:D + 2]
            m_n = jnp.maximum(m_a, m_b)
            alpha = jnp.exp(m_a - m_n)
            beta = jnp.exp(m_b - m_n)
            l_n = alpha * l_a + beta * l_b
            o_ref[:, 0, :, :] = o_a * (alpha / l_n) + o_b * (beta / l_n)

    return pl.pallas_call(
        c_body,
        grid=(B,),
        out_shape=jax.ShapeDtypeStruct((B, 1, H, D), jnp.float32),
        in_specs=[
            pl.BlockSpec((1, H, SKV), lambda b: (b, 0, 0)),
            pl.BlockSpec((1, H, 2), lambda b: (b, 0, 0)),
            pl.BlockSpec((1, H, D, SKV), lambda b: (b, 0, 0, 0)),
        ],
        out_specs=pl.BlockSpec((B, 1, H, D), lambda b: (0, 0, 0, 0)),
        scratch_shapes=[
            pltpu.VMEM((2, B, H, D + 2), jnp.float32),
            pltpu.SemaphoreType.DMA((NCHUNK,)),
            pltpu.SemaphoreType.DMA((NCHUNK,)),
        ],
        compiler_params=pltpu.CompilerParams(collective_id=0),
    )(P, stats, Vt)


# device time: 14527 ns/iter; 1.0496x vs baseline; 1.0496x over previous
import jax
import jax.numpy as jnp
from jax import lax
from jax.experimental import pallas as pl
from jax.experimental.pallas import tpu as pltpu

B = 8
SKV = 512
H = 8
D = 64
HD = H * D
SCALE = D ** -0.5
NCHUNK = 2
BC = B // NCHUNK


def _hd_mask():
    h2 = lax.broadcasted_iota(jnp.int32, (H, HD), 0)
    j2 = lax.broadcasted_iota(jnp.int32, (H, HD), 1)
    return (j2 // D == h2).astype(jnp.float32)


def kernel(Q, K, V):
    Kt = jnp.transpose(K, (0, 2, 3, 1))
    Vt = jnp.transpose(V, (0, 2, 3, 1))

    def s_body(q_ref, kt_ref, p_ref, stat_ref):
        hd_mask = _hd_mask()
        for b in range(B):
            qb = q_ref[b, 0]
            qrow = jnp.tile(qb, (1, H))
            wt = qrow * hd_mask * SCALE
            ktb = kt_ref[b].reshape(HD, SKV)
            sb = jnp.dot(wt, ktb, preferred_element_type=jnp.float32)
            mb = jnp.max(sb, axis=1, keepdims=True)
            pb = jnp.exp(sb - mb)
            lb = jnp.sum(pb, axis=1, keepdims=True)
            p_ref[b] = pb
            stat_ref[b, :, 0:1] = mb
            stat_ref[b, :, 1:2] = lb

    P, stats = pl.pallas_call(
        s_body,
        out_shape=(
            jax.ShapeDtypeStruct((B, H, SKV), jnp.float32),
            jax.ShapeDtypeStruct((B, H, 2), jnp.float32),
        ),
        in_specs=[pl.BlockSpec(memory_space=pltpu.VMEM)] * 2,
        out_specs=(pl.BlockSpec(memory_space=pltpu.VMEM),) * 2,
    )(Q, Kt)

    def c_body(p_ref, stat_ref, vt_ref, o_ref, comm, send_sems, recv_sems):
        my_x = lax.axis_index("x")
        my_y = lax.axis_index("y")
        peer = (my_x, 1 - my_y)

        barrier_sem = pltpu.get_barrier_semaphore()
        pl.semaphore_signal(barrier_sem, inc=1, device_id=peer,
                            device_id_type=pl.DeviceIdType.MESH)

        hd_mask = _hd_mask()
        j3 = lax.broadcasted_iota(jnp.int32, (HD, D), 0)
        d3 = lax.broadcasted_iota(jnp.int32, (HD, D), 1)
        e_sum = (j3 % D == d3).astype(jnp.float32)

        def rdma_chunk(c):
            lo = c * BC
            return pltpu.make_async_remote_copy(
                src_ref=comm.at[0, lo:lo + BC],
                dst_ref=comm.at[1, lo:lo + BC],
                send_sem=send_sems.at[c], recv_sem=recv_sems.at[c],
                device_id=peer, device_id_type=pl.DeviceIdType.MESH,
            )

        rdmas = []
        for c in range(NCHUNK):
            for b in range(c * BC, (c + 1) * BC):
                pb = p_ref[b]
                vtb = vt_ref[b].reshape(HD, SKV)
                of = lax.dot_general(pb, vtb, (((1,), (1,)), ((), ())),
                                     preferred_element_type=jnp.float32)
                ob = jnp.dot(of * hd_mask, e_sum,
                             preferred_element_type=jnp.float32)
                comm[0, b, :, 0:D] = ob
                comm[0, b, :, D:D + 2] = stat_ref[b]
            if c == 0:
                pl.semaphore_wait(barrier_sem, 1)
            r = rdma_chunk(c)
            r.start()
            rdmas.append(r)
        for r in rdmas:
            r.wait()

        o_a = comm[0, :, :, 0:D]
        m_a = comm[0, :, :, D:D + 1]
        l_a = comm[0, :, :, D + 1:D + 2]
        o_b = comm[1, :, :, 0:D]
        m_b = comm[1, :, :, D:D + 1]
        l_b = comm[1, :, :, D + 1:D + 2]
        m_n = jnp.maximum(m_a, m_b)
        alpha = jnp.exp(m_a - m_n)
        beta = jnp.exp(m_b - m_n)
        l_n = alpha * l_a + beta * l_b
        o_ref[:, 0, :, :] = o_a * (alpha / l_n) + o_b * (beta / l_n)

    return pl.pallas_call(
        c_body,
        out_shape=jax.ShapeDtypeStruct((B, 1, H, D), jnp.float32),
        in_specs=[pl.BlockSpec(memory_space=pltpu.VMEM)] * 3,
        out_specs=pl.BlockSpec(memory_space=pltpu.VMEM),
        scratch_shapes=[
            pltpu.VMEM((2, B, H, D + 2), jnp.float32),
            pltpu.SemaphoreType.DMA((NCHUNK,)),
            pltpu.SemaphoreType.DMA((NCHUNK,)),
        ],
        compiler_params=pltpu.CompilerParams(collective_id=0),
    )(P, stats, Vt)
